# baseline (speedup 1.0000x reference)
import functools

import jax
import jax.numpy as jnp
from jax.experimental import pallas as pl
from jax.experimental.pallas import tpu as pltpu

_BETA = 0.25


def _stage_kernel(q, r_ref, rs_ref, c2_ref, cb_ref,
                  dist_ref, idx_ref, xres_ref, rnext_ref, loss_ref,
                  cb1_ref, cb2_ref, cb3_ref):
    del q
    n_e, e_dim = cb_ref.shape
    bb = r_ref.shape[0]
    f32 = jnp.float32
    bf16 = jnp.bfloat16

    @pl.when(pl.program_id(0) == 0)
    def _precompute():
        cb = cb_ref[...]
        c1 = cb.astype(bf16)
        e1 = cb - c1.astype(f32)
        c2s = e1.astype(bf16)
        e2 = e1 - c2s.astype(f32)
        cb1_ref[...] = c1
        cb2_ref[...] = c2s
        cb3_ref[...] = e2.astype(bf16)

    r = r_ref[...]
    cross2 = jax.lax.dot_general(
        (r + r).astype(bf16), cb1_ref[...], (((1,), (1,)), ((), ())),
        preferred_element_type=f32)
    d = rs_ref[...] - cross2 + c2_ref[...]
    dist_ref[...] = d
    iota = jax.lax.broadcasted_iota(jnp.int32, (bb, n_e), 1)
    dmin = jnp.min(d, axis=1, keepdims=True)
    idx = jnp.min(jnp.where(d == dmin, iota, jnp.int32(n_e)), axis=1)
    idx_ref[...] = idx[:, None]
    one_hot = (iota == idx[:, None]).astype(f32).astype(bf16)
    gat = lambda c_part: jax.lax.dot_general(
        one_hot, c_part, (((1,), (0,)), ((), ())),
        preferred_element_type=f32)
    xq_rows = (gat(cb1_ref[...]) + gat(cb2_ref[...])) + gat(cb3_ref[...])
    xres = r + (xq_rows - r)
    xres_ref[...] = xres
    rnext_ref[...] = r - xres
    diff = xq_rows - r
    part = jnp.sum(diff * diff)
    prev = jnp.where(pl.program_id(0) == 0, jnp.zeros((1, 1), f32),
                     loss_ref[...])
    loss_ref[...] = prev + part


def _stage_call(q, num_q, r, rs, c2, cb, dist_buf):
    b, e_dim = r.shape
    n_e = cb.shape[0]
    bb = min(256, b)
    grid = b // bb
    kern = functools.partial(_stage_kernel, q)

    def _wrapped(r_ref, rs_ref, c2_ref, cb_ref, dist_in_ref, *rest):
        del dist_in_ref
        kern(r_ref, rs_ref, c2_ref, cb_ref, *rest)

    any_spec = pl.BlockSpec(memory_space=pl.ANY)
    in_specs = [
        pl.BlockSpec((bb, e_dim), lambda i: (i, 0)),
        pl.BlockSpec((bb, 1), lambda i: (i, 0)),
        pl.BlockSpec((1, n_e), lambda i: (0, 0)),
        pl.BlockSpec((n_e, e_dim), lambda i: (0, 0)),
    ]
    args = [r, rs, c2, cb]
    aliases = {}
    body = kern
    if q > 0:
        in_specs += [any_spec]
        args += [dist_buf]
        aliases = {4: 0}
        body = _wrapped
    out_specs = (
        pl.BlockSpec((bb, n_e), lambda i: (i, q)),
        pl.BlockSpec((bb, 1), lambda i: (i, 0)),
        pl.BlockSpec((bb, e_dim), lambda i: (i, 0)),
        pl.BlockSpec((bb, e_dim), lambda i: (i, 0)),
        pl.BlockSpec((1, 1), lambda i: (0, 0)),
    )
    out_shapes = (
        jax.ShapeDtypeStruct((b, num_q * n_e), jnp.float32),
        jax.ShapeDtypeStruct((b, 1), jnp.int32),
        jax.ShapeDtypeStruct((b, e_dim), jnp.float32),
        jax.ShapeDtypeStruct((b, e_dim), jnp.float32),
        jax.ShapeDtypeStruct((1, 1), jnp.float32),
    )
    scratch_shapes = [
        pltpu.VMEM((n_e, e_dim), jnp.bfloat16),
        pltpu.VMEM((n_e, e_dim), jnp.bfloat16),
        pltpu.VMEM((n_e, e_dim), jnp.bfloat16),
    ]
    return pl.pallas_call(
        body,
        grid=(grid,),
        in_specs=in_specs,
        out_specs=out_specs,
        out_shape=out_shapes,
        scratch_shapes=scratch_shapes,
        input_output_aliases=aliases,
    )(*args)


def kernel(x, codebooks):
    b, e_dim = x.shape
    num_q, n_e, _ = codebooks.shape
    residual = x
    x_q = None
    dist_buf = None
    idx_cols = []
    losses = []
    for q in range(num_q):
        rs = jnp.sum(residual ** 2, axis=1, keepdims=True)
        c2 = jnp.sum(codebooks[q] ** 2, axis=1)[None, :]
        dist_buf, idx_col, xres, residual, loss_sum = _stage_call(
            q, num_q, residual, rs, c2, codebooks[q], dist_buf)
        idx_cols.append(idx_col)
        m = loss_sum.reshape(()) / (b * e_dim)
        losses.append(m + _BETA * m)
        x_q = xres if x_q is None else x_q + xres
    mean_losses = jnp.stack(losses).mean()
    all_indices = jnp.concatenate(idx_cols, axis=1)
    all_distances = dist_buf.reshape(b, num_q, n_e)
    return x_q, mean_losses, all_indices, all_distances


# speedup vs baseline: 1.0217x; 1.0217x over previous
import functools

import jax
import jax.numpy as jnp
from jax.experimental import pallas as pl
from jax.experimental.pallas import tpu as pltpu

_BETA = 0.25


def _stage_kernel(q, num_q, r_ref, rs_ref, c2_ref, cb_ref, x_ref,
                  dist_ref, idx_ref, rnext_ref, loss_ref,
                  cb1_ref, cb2_ref, cb3_ref):
    last = q == num_q - 1
    del q
    n_e, e_dim = cb_ref.shape
    bb = r_ref.shape[0]
    f32 = jnp.float32
    bf16 = jnp.bfloat16

    @pl.when(pl.program_id(0) == 0)
    def _precompute():
        cb = cb_ref[...]
        c1 = cb.astype(bf16)
        e1 = cb - c1.astype(f32)
        c2s = e1.astype(bf16)
        e2 = e1 - c2s.astype(f32)
        cb1_ref[...] = c1
        cb2_ref[...] = c2s
        cb3_ref[...] = e2.astype(bf16)

    r = r_ref[...]
    cross2 = jax.lax.dot_general(
        (r + r).astype(bf16), cb1_ref[...], (((1,), (1,)), ((), ())),
        preferred_element_type=f32)
    d = rs_ref[...] - cross2 + c2_ref[...]
    dist_ref[...] = d
    iota = jax.lax.broadcasted_iota(jnp.int32, (bb, n_e), 1)
    dmin = jnp.min(d, axis=1, keepdims=True)
    idx = jnp.min(jnp.where(d == dmin, iota, jnp.int32(n_e)), axis=1)
    idx_ref[...] = idx[:, None]
    one_hot = (iota == idx[:, None]).astype(f32).astype(bf16)
    gat = lambda c_part: jax.lax.dot_general(
        one_hot, c_part, (((1,), (0,)), ((), ())),
        preferred_element_type=f32)
    xq_rows = (gat(cb1_ref[...]) + gat(cb2_ref[...])) + gat(cb3_ref[...])
    xres = r + (xq_rows - r)
    rnext = r - xres
    if last:
        rnext_ref[...] = x_ref[...] - rnext
    else:
        rnext_ref[...] = rnext
    diff = xq_rows - r
    part = jnp.sum(diff * diff)
    prev = jnp.where(pl.program_id(0) == 0, jnp.zeros((1, 1), f32),
                     loss_ref[...])
    loss_ref[...] = prev + part


def _stage_call(q, num_q, r, rs, c2, cb, x, dist_buf):
    b, e_dim = r.shape
    n_e = cb.shape[0]
    bb = min(256, b)
    grid = b // bb
    last = q == num_q - 1
    kern = functools.partial(_stage_kernel, q, num_q)

    row_spec = pl.BlockSpec((bb, e_dim), lambda i: (i, 0))
    any_spec = pl.BlockSpec(memory_space=pl.ANY)
    in_specs = [
        row_spec,
        pl.BlockSpec((bb, 1), lambda i: (i, 0)),
        pl.BlockSpec((1, n_e), lambda i: (0, 0)),
        pl.BlockSpec((n_e, e_dim), lambda i: (0, 0)),
    ]
    args = [r, rs, c2, cb]
    if last:
        in_specs.append(row_spec)
        args.append(x)

    def body(*refs):
        n_in = len(args)
        in_refs = list(refs[:n_in])
        out_refs = refs[n_in:]
        if q > 0:
            in_refs.pop()
        if not last:
            in_refs.append(None)
        kern(*in_refs, *out_refs)

    aliases = {}
    if q > 0:
        in_specs.append(any_spec)
        args.append(dist_buf)
        aliases = {len(args) - 1: 0}
    out_specs = (
        pl.BlockSpec((bb, n_e), lambda i: (i, q)),
        pl.BlockSpec((bb, 1), lambda i: (i, 0)),
        pl.BlockSpec((bb, e_dim), lambda i: (i, 0)),
        pl.BlockSpec((1, 1), lambda i: (0, 0)),
    )
    out_shapes = (
        jax.ShapeDtypeStruct((b, num_q * n_e), jnp.float32),
        jax.ShapeDtypeStruct((b, 1), jnp.int32),
        jax.ShapeDtypeStruct((b, e_dim), jnp.float32),
        jax.ShapeDtypeStruct((1, 1), jnp.float32),
    )
    scratch_shapes = [
        pltpu.VMEM((n_e, e_dim), jnp.bfloat16),
        pltpu.VMEM((n_e, e_dim), jnp.bfloat16),
        pltpu.VMEM((n_e, e_dim), jnp.bfloat16),
    ]
    return pl.pallas_call(
        body,
        grid=(grid,),
        in_specs=in_specs,
        out_specs=out_specs,
        out_shape=out_shapes,
        scratch_shapes=scratch_shapes,
        input_output_aliases=aliases,
    )(*args)


def kernel(x, codebooks):
    b, e_dim = x.shape
    num_q, n_e, _ = codebooks.shape
    residual = x
    dist_buf = None
    idx_cols = []
    losses = []
    for q in range(num_q):
        rs = jnp.sum(residual ** 2, axis=1, keepdims=True)
        c2 = jnp.sum(codebooks[q] ** 2, axis=1)[None, :]
        dist_buf, idx_col, residual, loss_sum = _stage_call(
            q, num_q, residual, rs, c2, codebooks[q], x, dist_buf)
        idx_cols.append(idx_col)
        m = loss_sum.reshape(()) / (b * e_dim)
        losses.append(m + _BETA * m)
    x_q = residual
    mean_losses = jnp.stack(losses).mean()
    all_indices = jnp.concatenate(idx_cols, axis=1)
    all_distances = dist_buf.reshape(b, num_q, n_e)
    return x_q, mean_losses, all_indices, all_distances


# speedup vs baseline: 1.7447x; 1.7076x over previous
import jax
import jax.numpy as jnp
from jax.experimental import pallas as pl
from jax.experimental.pallas import tpu as pltpu

_BETA = 0.25


def _row_norm_sq(r):
    e = r.shape[1]
    x2 = r * r
    h = e // 2
    y = x2[:, :h] + x2[:, h:]
    yt = y.T
    acc = yt[0:8]
    for k in range(1, h // 8):
        acc = acc + yt[8 * k : 8 * k + 8]
    t = acc[0:4] + acc[4:8]
    u = t[0:2] + t[2:4]
    rs_row = u[0:1] + u[1:2]
    return rs_row.T


def _rvq_kernel(x_ref, cb_ref, xq_ref, idx_ref, loss_ref, dist_ref,
                c2_ref, cb1_ref, cb2_ref, cb3_ref):
    num_q, n_e, e_dim = cb_ref.shape
    bb = x_ref.shape[0]
    f32 = jnp.float32
    bf16 = jnp.bfloat16

    @pl.when(pl.program_id(0) == 0)
    def _precompute():
        for q in range(num_q):
            cb = cb_ref[q]
            c1 = cb.astype(bf16)
            e1 = cb - c1.astype(f32)
            c2s = e1.astype(bf16)
            e2 = e1 - c2s.astype(f32)
            cb1_ref[q] = c1
            cb2_ref[q] = c2s
            cb3_ref[q] = e2.astype(bf16)
            c2_ref[q] = _row_norm_sq(cb).T

    r = x_ref[...]
    xq_acc = jnp.zeros_like(r)
    loss_acc = jnp.float32(0.0)
    idx_cols = []
    for q in range(num_q):
        rs = _row_norm_sq(r)
        cross2 = jax.lax.dot_general(
            (r + r).astype(bf16), cb1_ref[q], (((1,), (1,)), ((), ())),
            preferred_element_type=f32)
        d = rs - cross2 + c2_ref[q]
        dist_ref[:, q, :] = d
        iota = jax.lax.broadcasted_iota(jnp.int32, (bb, n_e), 1)
        dmin = jnp.min(d, axis=1, keepdims=True)
        idx = jnp.min(jnp.where(d == dmin, iota, jnp.int32(n_e)), axis=1)
        idx_cols.append(idx)
        one_hot = (iota == idx[:, None]).astype(f32).astype(bf16)
        gat = lambda c_part: jax.lax.dot_general(
            one_hot, c_part, (((1,), (0,)), ((), ())),
            preferred_element_type=f32)
        xq_rows = (gat(cb1_ref[q]) + gat(cb2_ref[q])) + gat(cb3_ref[q])
        xres = r + (xq_rows - r)
        diff = xq_rows - r
        loss_acc = loss_acc + jnp.sum(diff * diff)
        xq_acc = xq_acc + xres
        r = r - xres
    xq_ref[...] = xq_acc
    idx_ref[...] = jnp.stack(idx_cols, axis=1)

    step = pl.program_id(0)
    total_b = pl.num_programs(0) * bb
    prev = jnp.where(step == 0, jnp.zeros((1, 1), f32), loss_ref[...])
    total = prev + loss_acc
    scale = (1.0 + _BETA) / (total_b * e_dim * num_q)
    loss_ref[...] = jnp.where(step == pl.num_programs(0) - 1,
                              total * scale, total)


def kernel(x, codebooks):
    b, e_dim = x.shape
    num_q, n_e, _ = codebooks.shape
    bb = min(256, b)
    grid = b // bb
    out_shapes = (
        jax.ShapeDtypeStruct((b, e_dim), jnp.float32),
        jax.ShapeDtypeStruct((b, num_q), jnp.int32),
        jax.ShapeDtypeStruct((1, 1), jnp.float32),
        jax.ShapeDtypeStruct((b, num_q, n_e), jnp.float32),
    )
    in_specs = [
        pl.BlockSpec((bb, e_dim), lambda i: (i, 0)),
        pl.BlockSpec((num_q, n_e, e_dim), lambda i: (0, 0, 0)),
    ]
    out_specs = (
        pl.BlockSpec((bb, e_dim), lambda i: (i, 0)),
        pl.BlockSpec((bb, num_q), lambda i: (i, 0)),
        pl.BlockSpec((1, 1), lambda i: (0, 0)),
        pl.BlockSpec((bb, num_q, n_e), lambda i: (i, 0, 0)),
    )
    scratch_shapes = [
        pltpu.VMEM((num_q, 1, n_e), jnp.float32),
        pltpu.VMEM((num_q, n_e, e_dim), jnp.bfloat16),
        pltpu.VMEM((num_q, n_e, e_dim), jnp.bfloat16),
        pltpu.VMEM((num_q, n_e, e_dim), jnp.bfloat16),
    ]
    xq, idx, loss, dist = pl.pallas_call(
        _rvq_kernel,
        grid=(grid,),
        in_specs=in_specs,
        out_specs=out_specs,
        out_shape=out_shapes,
        scratch_shapes=scratch_shapes,
    )(x, codebooks)
    return xq, loss.reshape(()), idx, dist
